# baseline (speedup 1.0000x reference)
import functools

import jax
import jax.numpy as jnp
from jax import lax
from jax.experimental import pallas as pl
from jax.experimental.pallas import tpu as pltpu
from jax.experimental.pallas import tpu_sc as plsc

NN = 10000
EE = 320000
DD = 128
HD = DD // 2
BB = 64
NCLS = 10
TEMP = 0.07
NEG_W = 0.8

NP_ = 10240
TRASH = NP_ - 1
NTILES = 32
CH = 128
NCH = 80
EPT = CH * NCH
EPAD = NTILES * EPT
RPT = NP_ // 16
MC = 2
ND = NCH // MC


@functools.cache
def _sc_degree_kernel():
    mesh = plsc.VectorSubcoreMesh(core_axis_name="c", subcore_axis_name="s")
    return functools.partial(
        pl.kernel, mesh=mesh,
        out_type=jax.ShapeDtypeStruct((64, NP_), jnp.float32),
        scratch_types=[
            pltpu.VMEM((ND, MC * CH), jnp.int32),
            pltpu.VMEM((NP_,), jnp.float32),
        ],
        compiler_params=pltpu.CompilerParams(needs_layout_passes=False),
    )(_sc_degree_body)


def _sc_degree_body(dst_hbm, out_hbm, dbuf, cnt):
    c = lax.axis_index("c")
    s = lax.axis_index("s")
    wid = s * 2 + c
    ones16 = jnp.ones((16,), jnp.float32)
    zero16 = jnp.zeros((16,), jnp.float32)
    for br in range(2):
        def zbody(i, _):
            cnt[pl.ds(i * 16, 16)] = zero16
            return 0
        lax.fori_loop(0, NP_ // 16, zbody, 0)
        pltpu.sync_copy(dst_hbm.at[pl.ds((br * NTILES + wid) * ND, ND)], dbuf)

        def chunk(t, _):
            for j in range((MC * CH) // 16):
                idx = dbuf[t, pl.ds(j * 16, 16)]
                plsc.addupdate_scatter(cnt, [idx], ones16)
            return 0
        lax.fori_loop(0, ND, chunk, 0)
        pltpu.sync_copy(cnt, out_hbm.at[br * NTILES + wid])


@functools.cache
def _sc_spmm_kernel():
    mesh = plsc.VectorSubcoreMesh(core_axis_name="c", subcore_axis_name="s")
    return functools.partial(
        pl.kernel, mesh=mesh,
        out_type=jax.ShapeDtypeStruct((8 * NP_, HD), jnp.float32),
        scratch_types=[
            pltpu.VMEM((ND, MC * CH), jnp.int32),
            pltpu.VMEM((ND, MC * CH), jnp.int32),
            pltpu.VMEM((MC * CH, HD), jnp.float32),
            pltpu.VMEM((MC * CH, HD), jnp.float32),
            pltpu.VMEM((CH, HD), jnp.float32),
            pltpu.VMEM_SHARED((NP_, HD), jnp.float32),
            pltpu.SemaphoreType.DMA,
            pltpu.SemaphoreType.DMA,
        ],
        compiler_params=pltpu.CompilerParams(
            needs_layout_passes=False, use_tc_tiling_on_sc=False),
    )(_sc_spmm_body)


def _sc_spmm_body(z0_hbm, z1_hbm, src_hbm, dst_hbm, out_hbm,
                  sb, db, s0, s1, zb, acc, sem, sem2):
    c = lax.axis_index("c")
    s = lax.axis_index("s")
    wid = s * 2 + c
    slots = [s0, s1]
    zero16 = jnp.zeros((16,), jnp.float32)

    def zrow(i, _):
        for j in range(HD // 16):
            zb[i, pl.ds(j * 16, 16)] = zero16
        return 0
    lax.fori_loop(0, CH, zrow, 0)

    for br in range(2):
        ebase = (br * NTILES + wid) * ND
        pltpu.sync_copy(src_hbm.at[pl.ds(ebase, ND)], sb)
        pltpu.sync_copy(dst_hbm.at[pl.ds(ebase, ND)], db)
        for h, zt in enumerate((z0_hbm, z1_hbm)):
            def zacc(k, _):
                pltpu.sync_copy(zb, acc.at[pl.ds(s * RPT + k * CH, CH)])
                return 0
            lax.fori_loop(0, RPT // CH, zacc, 0)
            plsc.subcore_barrier()

            pltpu.async_copy(zt.at[sb.at[0]], slots[0], sem)

            def sup(u, _):
                for par in range(2):
                    g = 2 * u + par
                    buf = slots[par]
                    obuf = slots[1 - par]
                    pltpu.make_async_copy(zt.at[sb.at[g]], buf, sem).wait()
                    pltpu.async_copy(buf, acc.at[db.at[g]], sem2, add=True)

                    @pl.when(g >= 1)
                    def _():
                        pltpu.make_async_copy(
                            obuf, acc.at[db.at[g]], sem2).wait()

                    @pl.when(g + 1 < ND)
                    def _():
                        pltpu.async_copy(zt.at[sb.at[g + 1]], obuf, sem)
                return 0
            lax.fori_loop(0, ND // 2, sup, 0)
            pltpu.make_async_copy(slots[1], acc.at[db.at[0]], sem2).wait()
            plsc.subcore_barrier()

            obase = (h * 4 + br * 2 + c) * NP_ + s * RPT
            pltpu.sync_copy(acc.at[pl.ds(s * RPT, RPT)],
                            out_hbm.at[pl.ds(obase, RPT)])
            plsc.subcore_barrier()


_NRB = NP_ // 512


def _dinv_body(cnt_ref, out_ref):
    i = pl.program_id(1)
    s = jnp.sum(cnt_ref[...], axis=0, keepdims=True)
    col = i * 128 + lax.broadcasted_iota(jnp.int32, (1, 128), 1)
    deg = s + jnp.where(col < NN, 1.0, 0.0)
    v = jnp.where(deg > 0, lax.rsqrt(deg), 0.0)
    out_ref[...] = jnp.broadcast_to(v, (128, 128)).T[:, :HD]


def _tc_dinv(cnt):
    return pl.pallas_call(
        _dinv_body,
        grid=(2, NP_ // 128),
        in_specs=[pl.BlockSpec((32, 128), lambda b, i: (b, i))],
        out_specs=pl.BlockSpec((128, HD), lambda b, i: (b * (NP_ // 128) + i, 0)),
        out_shape=jax.ShapeDtypeStruct((2 * NP_, HD), jnp.float32),
    )(cnt)


_HP = dict(preferred_element_type=jnp.float32,
           precision=jax.lax.Precision.HIGHEST)
_DN = (((1,), (0,)), ((), ()))
_DT = (((1,), (1,)), ((), ()))
_DC = (((0,), (0,)), ((), ()))

_ZSPEC = pl.BlockSpec((512, HD), lambda b, i: (b * _NRB + i, 0))


def _p_spec(c, h):
    return pl.BlockSpec((512, HD), lambda b, i, c=c, h=h:
                        ((h * 4 + b * 2 + c) * _NRB + i, 0))


def _prep_body(x_ref, w_ref, dv_ref, o0_ref, o1_ref):
    h = jax.lax.dot_general(x_ref[...], w_ref[0], _DN, **_HP)
    dv = dv_ref[...]
    o0_ref[...] = h[:, :HD] * dv
    o1_ref[...] = h[:, HD:] * dv


def _tc_prep(xs, ws, dinv):
    return pl.pallas_call(
        _prep_body,
        grid=(2, _NRB),
        in_specs=[
            pl.BlockSpec((512, 128), lambda b, i: (b * _NRB + i, 0)),
            pl.BlockSpec((1, 128, 128), lambda b, i: (b, 0, 0)),
            _ZSPEC,
        ],
        out_specs=[_ZSPEC, _ZSPEC],
        out_shape=[jax.ShapeDtypeStruct((2 * NP_, HD), jnp.float32),
                   jax.ShapeDtypeStruct((2 * NP_, HD), jnp.float32)],
    )(xs, ws, dinv)


def _mid_body(p00_ref, p10_ref, p01_ref, p11_ref, z0_ref, z1_ref, dv_ref,
              b_ref, w_ref, o0_ref, o1_ref):
    dv = dv_ref[...]
    b = b_ref[0]
    w = w_ref[0]
    y0 = jnp.maximum((p00_ref[...] + p10_ref[...] + z0_ref[...]) * dv
                     + b[:, :HD], 0.0)
    y1 = jnp.maximum((p01_ref[...] + p11_ref[...] + z1_ref[...]) * dv
                     + b[:, HD:], 0.0)
    h2 = (jax.lax.dot_general(y0, w[:HD, :], _DN, **_HP)
          + jax.lax.dot_general(y1, w[HD:, :], _DN, **_HP))
    o0_ref[...] = h2[:, :HD] * dv
    o1_ref[...] = h2[:, HD:] * dv


def _tc_mid(partials, z0, z1, dinv, bs, ws):
    return pl.pallas_call(
        _mid_body,
        grid=(2, _NRB),
        in_specs=[
            _p_spec(0, 0), _p_spec(1, 0), _p_spec(0, 1), _p_spec(1, 1),
            _ZSPEC, _ZSPEC, _ZSPEC,
            pl.BlockSpec((1, 1, 128), lambda b, i: (b, 0, 0)),
            pl.BlockSpec((1, 128, 128), lambda b, i: (b, 0, 0)),
        ],
        out_specs=[_ZSPEC, _ZSPEC],
        out_shape=[jax.ShapeDtypeStruct((2 * NP_, HD), jnp.float32),
                   jax.ShapeDtypeStruct((2 * NP_, HD), jnp.float32)],
    )(partials, partials, partials, partials, z0, z1, dinv, bs, ws)


def _pool_body(p00_ref, p10_ref, p01_ref, p11_ref, z0_ref, z1_ref, dv_ref,
               b_ref, batch_ref, o0_ref, o1_ref):
    i = pl.program_id(1)
    dv = dv_ref[...]
    b = b_ref[0]
    y0 = jnp.maximum((p00_ref[...] + p10_ref[...] + z0_ref[...]) * dv
                     + b[:, :HD], 0.0)
    y1 = jnp.maximum((p01_ref[...] + p11_ref[...] + z1_ref[...]) * dv
                     + b[:, HD:], 0.0)
    seg = lax.broadcasted_iota(jnp.int32, (512, BB), 1)
    p = jnp.where(batch_ref[...] == seg, 1.0, 0.0)
    a0 = jax.lax.dot_general(p, y0, _DC, **_HP)
    a1 = jax.lax.dot_general(p, y1, _DC, **_HP)

    @pl.when(i == 0)
    def _():
        o0_ref[...] = jnp.zeros_like(o0_ref)
        o1_ref[...] = jnp.zeros_like(o1_ref)
    o0_ref[...] += a0
    o1_ref[...] += a1


def _tc_pool(partials, z0, z1, dinv, bs, batch2d):
    ospec = pl.BlockSpec((BB, HD), lambda b, i: (b, 0))
    return pl.pallas_call(
        _pool_body,
        grid=(2, _NRB),
        in_specs=[
            _p_spec(0, 0), _p_spec(1, 0), _p_spec(0, 1), _p_spec(1, 1),
            _ZSPEC, _ZSPEC, _ZSPEC,
            pl.BlockSpec((1, 1, 128), lambda b, i: (b, 0, 0)),
            pl.BlockSpec((512, 1), lambda b, i: (i, 0)),
        ],
        out_specs=[ospec, ospec],
        out_shape=[jax.ShapeDtypeStruct((2 * BB, HD), jnp.float32),
                   jax.ShapeDtypeStruct((2 * BB, HD), jnp.float32)],
    )(partials, partials, partials, partials, z0, z1, dinv, bs, batch2d)


def _head_body(sp_ref, fp_ref, w1a_ref, w1b_ref, b1_ref, w2_ref, b2_ref,
               out_ref):
    sp = sp_ref[...]
    fp = fp_ref[...]
    x1 = (jax.lax.dot_general(sp, w1a_ref[...], _DN, **_HP)
          + jax.lax.dot_general(fp, w1b_ref[...], _DN, **_HP) + b1_ref[...])
    x1 = jnp.maximum(x1, 0.0)
    lg = jax.lax.dot_general(x1, w2_ref[...], _DN, **_HP) + b2_ref[...]
    m = jnp.max(lg, axis=1, keepdims=True)
    logp = lg - m - jnp.log(jnp.sum(jnp.exp(lg - m), axis=1, keepdims=True))

    def nrm(v):
        n = jnp.sqrt(jnp.sum(v * v, axis=1, keepdims=True))
        return v / jnp.maximum(n, 1e-12)

    scn = nrm(sp)
    fcn = nrm(fp)
    row = lax.broadcasted_iota(jnp.int32, (BB, BB), 0)
    col = lax.broadcasted_iota(jnp.int32, (BB, BB), 1)
    off = jnp.where(row == col, 0.0, 1.0)
    sf = jax.lax.dot_general(scn, fcn, _DT, **_HP) / TEMP
    fs = jax.lax.dot_general(fcn, scn, _DT, **_HP) / TEMP
    ssg = jax.lax.dot_general(scn, scn, _DT, **_HP) / TEMP * NEG_W * off
    ffg = jax.lax.dot_general(fcn, fcn, _DT, **_HP) / TEMP * NEG_W * off
    diag = jnp.sum(jnp.where(row == col, sf, 0.0), axis=1, keepdims=True)

    def lse2(a, bmat):
        mm = jnp.maximum(jnp.max(a, axis=1, keepdims=True),
                         jnp.max(bmat, axis=1, keepdims=True))
        ssum = (jnp.sum(jnp.exp(a - mm), axis=1, keepdims=True)
                + jnp.sum(jnp.exp(bmat - mm), axis=1, keepdims=True))
        return mm + jnp.log(ssum)

    loss_i = jnp.sum(lse2(sf, ssg) - diag) / BB
    loss_t = jnp.sum(lse2(fs, ffg) - diag) / BB
    out_ref[...] = logp + (loss_i + loss_t) * 0.5


def _tc_head(pooled, W1, b1, W2, b2):
    return pl.pallas_call(
        _head_body,
        out_shape=jax.ShapeDtypeStruct((BB, NCLS), jnp.float32),
    )(pooled[:BB], pooled[BB:], W1[:128], W1[128:], b1, W2, b2)


def _pack_edges(ei, z_off):
    src = jnp.concatenate([ei[0] + z_off,
                           jnp.full((EPAD - EE,), z_off, jnp.int32)])
    dst = jnp.concatenate([ei[1], jnp.full((EPAD - EE,), TRASH, jnp.int32)])
    return (src.reshape(NTILES * ND, MC * CH),
            dst.reshape(NTILES * ND, MC * CH))


def kernel(sc_x, sc_edge_index, batch, fc_x, fc_edge_index, Wsc0, bsc0, Wsc1,
           bsc1, Wfc0, bfc0, Wfc1, bfc1, W1, b1, W2, b2):
    pad = jnp.zeros((NP_ - NN, DD), jnp.float32)
    xs = jnp.concatenate([sc_x, pad, fc_x, pad], axis=0)
    ssrc, sdst = _pack_edges(sc_edge_index, 0)
    fsrc, fdst = _pack_edges(fc_edge_index, NP_)
    src2d = jnp.concatenate([ssrc, fsrc], axis=0)
    dst2d = jnp.concatenate([sdst, fdst], axis=0)
    batch2d = jnp.concatenate(
        [batch, jnp.full((NP_ - NN,), BB, jnp.int32)]).reshape(NP_, 1)

    w0s = jnp.stack([Wsc0, Wfc0])
    w1s = jnp.stack([Wsc1, Wfc1])
    b0s = jnp.stack([bsc0.reshape(1, DD), bfc0.reshape(1, DD)])
    b1s = jnp.stack([bsc1.reshape(1, DD), bfc1.reshape(1, DD)])

    cnt = _sc_degree_kernel()(dst2d)
    dinv = _tc_dinv(cnt)
    za, zb = _tc_prep(xs, w0s, dinv)
    p0 = _sc_spmm_kernel()(za, zb, src2d, dst2d)
    za1, zb1 = _tc_mid(p0, za, zb, dinv, b0s, w1s)
    p1 = _sc_spmm_kernel()(za1, zb1, src2d, dst2d)
    ph0, ph1 = _tc_pool(p1, za1, zb1, dinv, b1s, batch2d)
    pooled = jnp.concatenate([ph0, ph1], axis=1)
    return _tc_head(pooled, W1, b1.reshape(1, DD), W2, b2.reshape(1, NCLS))


# speedup vs baseline: 1.0973x; 1.0973x over previous
import functools

import jax
import jax.numpy as jnp
from jax import lax
from jax.experimental import pallas as pl
from jax.experimental.pallas import tpu as pltpu
from jax.experimental.pallas import tpu_sc as plsc

NN = 10000
EE = 320000
DD = 128
HD = DD // 2
BB = 64
NCLS = 10
TEMP = 0.07
NEG_W = 0.8

NP_ = 10240
TRASH = NP_ - 1
NTILES = 32
CH = 128
NCH = 80
EPT = CH * NCH
EPAD = NTILES * EPT
RPT = NP_ // 16
NB = 4


@functools.cache
def _sc_degree_kernel():
    mesh = plsc.VectorSubcoreMesh(core_axis_name="c", subcore_axis_name="s")
    return functools.partial(
        pl.kernel, mesh=mesh,
        out_type=jax.ShapeDtypeStruct((64, NP_), jnp.float32),
        scratch_types=[
            pltpu.VMEM((NCH, CH), jnp.int32),
            pltpu.VMEM((NP_,), jnp.float32),
        ],
        compiler_params=pltpu.CompilerParams(needs_layout_passes=False),
    )(_sc_degree_body)


def _sc_degree_body(dst_hbm, out_hbm, dbuf, cnt):
    c = lax.axis_index("c")
    s = lax.axis_index("s")
    wid = s * 2 + c
    ones16 = jnp.ones((16,), jnp.float32)
    zero16 = jnp.zeros((16,), jnp.float32)
    for br in range(2):
        def zbody(i, _):
            cnt[pl.ds(i * 16, 16)] = zero16
            return 0
        lax.fori_loop(0, NP_ // 16, zbody, 0)
        pltpu.sync_copy(dst_hbm.at[pl.ds((br * NTILES + wid) * NCH, NCH)], dbuf)

        def chunk(t, _):
            for j in range(CH // 16):
                idx = dbuf[t, pl.ds(j * 16, 16)]
                plsc.addupdate_scatter(cnt, [idx], ones16)
            return 0
        lax.fori_loop(0, NCH, chunk, 0)
        pltpu.sync_copy(cnt, out_hbm.at[br * NTILES + wid])


@functools.cache
def _sc_spmm_kernel():
    mesh = plsc.VectorSubcoreMesh(core_axis_name="c", subcore_axis_name="s")
    return functools.partial(
        pl.kernel, mesh=mesh,
        out_type=jax.ShapeDtypeStruct((8 * NP_, HD), jnp.float32),
        scratch_types=[
            pltpu.VMEM((NCH, CH), jnp.int32),
            pltpu.VMEM((NCH, CH), jnp.int32),
        ] + [pltpu.VMEM((CH, HD), jnp.float32) for _ in range(2 * NB)] + [
            pltpu.VMEM_SHARED((NP_, HD), jnp.float32),
            pltpu.SemaphoreType.DMA,
            pltpu.SemaphoreType.DMA,
        ],
        compiler_params=pltpu.CompilerParams(
            needs_layout_passes=False, use_tc_tiling_on_sc=False),
    )(_sc_spmm_body)


def _sc_spmm_body(z0_hbm, z1_hbm, src_hbm, dst_hbm, out_hbm,
                  sb, db, r0, r1, r2, r3, r4, r5, r6, r7, acc, sem, sem2):
    c = lax.axis_index("c")
    s = lax.axis_index("s")
    wid = s * 2 + c
    rows = [r0, r1, r2, r3, r4, r5, r6, r7]
    zero16 = jnp.zeros((16,), jnp.float32)

    def zrow(i, _):
        for j in range(HD // 16):
            r0[i, pl.ds(j * 16, 16)] = zero16
        return 0
    lax.fori_loop(0, CH, zrow, 0)

    for br in range(2):
        ebase = (br * NTILES + wid) * NCH
        pltpu.sync_copy(src_hbm.at[pl.ds(ebase, NCH)], sb)
        pltpu.sync_copy(dst_hbm.at[pl.ds(ebase, NCH)], db)
        for h, zt in enumerate((z0_hbm, z1_hbm)):
            def zrow2(i, _):
                for j in range(HD // 16):
                    r0[i, pl.ds(j * 16, 16)] = zero16
                return 0
            lax.fori_loop(0, CH, zrow2, 0)

            def zacc(k, _):
                pltpu.sync_copy(r0, acc.at[pl.ds(s * RPT + k * CH, CH)])
                return 0
            lax.fori_loop(0, RPT // CH, zacc, 0)
            plsc.subcore_barrier()

            for b in range(NB):
                pltpu.async_copy(zt.at[sb.at[b]], rows[b], sem)

            def sup(u, _):
                for par in range(2):
                    for b in range(NB):
                        t = (2 * u + par) * NB + b
                        buf = rows[par * NB + b]
                        obuf = rows[(1 - par) * NB + b]
                        pltpu.make_async_copy(
                            zt.at[sb.at[t]], buf, sem).wait()
                        pltpu.async_copy(
                            buf, acc.at[db.at[t]], sem2, add=True)
                        pt = t - NB

                        @pl.when(pt >= 0)
                        def _():
                            pltpu.make_async_copy(
                                obuf, acc.at[db.at[t]], sem2).wait()
                        nt = t + NB

                        @pl.when(nt < NCH)
                        def _():
                            pltpu.async_copy(zt.at[sb.at[nt]], obuf, sem)
                return 0
            lax.fori_loop(0, NCH // (2 * NB), sup, 0)
            for b in range(NB):
                pltpu.make_async_copy(rows[b], acc.at[db.at[0]], sem2).wait()
            plsc.subcore_barrier()

            obase = (h * 4 + br * 2 + c) * NP_ + s * RPT
            pltpu.sync_copy(acc.at[pl.ds(s * RPT, RPT)],
                            out_hbm.at[pl.ds(obase, RPT)])
            plsc.subcore_barrier()


_NRB = NP_ // 512


def _dinv_body(cnt_ref, out_ref):
    i = pl.program_id(1)
    s = jnp.sum(cnt_ref[...], axis=0, keepdims=True)
    col = i * 128 + lax.broadcasted_iota(jnp.int32, (1, 128), 1)
    deg = s + jnp.where(col < NN, 1.0, 0.0)
    v = jnp.where(deg > 0, lax.rsqrt(deg), 0.0)
    out_ref[...] = jnp.broadcast_to(v, (128, 128)).T[:, :HD]


def _tc_dinv(cnt):
    return pl.pallas_call(
        _dinv_body,
        grid=(2, NP_ // 128),
        in_specs=[pl.BlockSpec((32, 128), lambda b, i: (b, i))],
        out_specs=pl.BlockSpec((128, HD), lambda b, i: (b * (NP_ // 128) + i, 0)),
        out_shape=jax.ShapeDtypeStruct((2 * NP_, HD), jnp.float32),
    )(cnt)


_HP = dict(preferred_element_type=jnp.float32,
           precision=jax.lax.Precision.HIGHEST)
_DN = (((1,), (0,)), ((), ()))
_DT = (((1,), (1,)), ((), ()))
_DC = (((0,), (0,)), ((), ()))

_ZSPEC = pl.BlockSpec((512, HD), lambda b, i: (b * _NRB + i, 0))


def _p_spec(c, h):
    return pl.BlockSpec((512, HD), lambda b, i, c=c, h=h:
                        ((h * 4 + b * 2 + c) * _NRB + i, 0))


def _prep_body(x_ref, w_ref, dv_ref, o0_ref, o1_ref):
    h = jax.lax.dot_general(x_ref[...], w_ref[0], _DN, **_HP)
    dv = dv_ref[...]
    o0_ref[...] = h[:, :HD] * dv
    o1_ref[...] = h[:, HD:] * dv


def _tc_prep(xs, ws, dinv):
    return pl.pallas_call(
        _prep_body,
        grid=(2, _NRB),
        in_specs=[
            pl.BlockSpec((512, 128), lambda b, i: (b * _NRB + i, 0)),
            pl.BlockSpec((1, 128, 128), lambda b, i: (b, 0, 0)),
            _ZSPEC,
        ],
        out_specs=[_ZSPEC, _ZSPEC],
        out_shape=[jax.ShapeDtypeStruct((2 * NP_, HD), jnp.float32),
                   jax.ShapeDtypeStruct((2 * NP_, HD), jnp.float32)],
    )(xs, ws, dinv)


def _mid_body(p00_ref, p10_ref, p01_ref, p11_ref, z0_ref, z1_ref, dv_ref,
              b_ref, w_ref, o0_ref, o1_ref):
    dv = dv_ref[...]
    b = b_ref[0]
    w = w_ref[0]
    y0 = jnp.maximum((p00_ref[...] + p10_ref[...] + z0_ref[...]) * dv
                     + b[:, :HD], 0.0)
    y1 = jnp.maximum((p01_ref[...] + p11_ref[...] + z1_ref[...]) * dv
                     + b[:, HD:], 0.0)
    h2 = (jax.lax.dot_general(y0, w[:HD, :], _DN, **_HP)
          + jax.lax.dot_general(y1, w[HD:, :], _DN, **_HP))
    o0_ref[...] = h2[:, :HD] * dv
    o1_ref[...] = h2[:, HD:] * dv


def _tc_mid(partials, z0, z1, dinv, bs, ws):
    return pl.pallas_call(
        _mid_body,
        grid=(2, _NRB),
        in_specs=[
            _p_spec(0, 0), _p_spec(1, 0), _p_spec(0, 1), _p_spec(1, 1),
            _ZSPEC, _ZSPEC, _ZSPEC,
            pl.BlockSpec((1, 1, 128), lambda b, i: (b, 0, 0)),
            pl.BlockSpec((1, 128, 128), lambda b, i: (b, 0, 0)),
        ],
        out_specs=[_ZSPEC, _ZSPEC],
        out_shape=[jax.ShapeDtypeStruct((2 * NP_, HD), jnp.float32),
                   jax.ShapeDtypeStruct((2 * NP_, HD), jnp.float32)],
    )(partials, partials, partials, partials, z0, z1, dinv, bs, ws)


def _pool_body(p00_ref, p10_ref, p01_ref, p11_ref, z0_ref, z1_ref, dv_ref,
               b_ref, batch_ref, o0_ref, o1_ref):
    i = pl.program_id(1)
    dv = dv_ref[...]
    b = b_ref[0]
    y0 = jnp.maximum((p00_ref[...] + p10_ref[...] + z0_ref[...]) * dv
                     + b[:, :HD], 0.0)
    y1 = jnp.maximum((p01_ref[...] + p11_ref[...] + z1_ref[...]) * dv
                     + b[:, HD:], 0.0)
    seg = lax.broadcasted_iota(jnp.int32, (512, BB), 1)
    p = jnp.where(batch_ref[...] == seg, 1.0, 0.0)
    a0 = jax.lax.dot_general(p, y0, _DC, **_HP)
    a1 = jax.lax.dot_general(p, y1, _DC, **_HP)

    @pl.when(i == 0)
    def _():
        o0_ref[...] = jnp.zeros_like(o0_ref)
        o1_ref[...] = jnp.zeros_like(o1_ref)
    o0_ref[...] += a0
    o1_ref[...] += a1


def _tc_pool(partials, z0, z1, dinv, bs, batch2d):
    ospec = pl.BlockSpec((BB, HD), lambda b, i: (b, 0))
    return pl.pallas_call(
        _pool_body,
        grid=(2, _NRB),
        in_specs=[
            _p_spec(0, 0), _p_spec(1, 0), _p_spec(0, 1), _p_spec(1, 1),
            _ZSPEC, _ZSPEC, _ZSPEC,
            pl.BlockSpec((1, 1, 128), lambda b, i: (b, 0, 0)),
            pl.BlockSpec((512, 1), lambda b, i: (i, 0)),
        ],
        out_specs=[ospec, ospec],
        out_shape=[jax.ShapeDtypeStruct((2 * BB, HD), jnp.float32),
                   jax.ShapeDtypeStruct((2 * BB, HD), jnp.float32)],
    )(partials, partials, partials, partials, z0, z1, dinv, bs, batch2d)


def _head_body(sp_ref, fp_ref, w1a_ref, w1b_ref, b1_ref, w2_ref, b2_ref,
               out_ref):
    sp = sp_ref[...]
    fp = fp_ref[...]
    x1 = (jax.lax.dot_general(sp, w1a_ref[...], _DN, **_HP)
          + jax.lax.dot_general(fp, w1b_ref[...], _DN, **_HP) + b1_ref[...])
    x1 = jnp.maximum(x1, 0.0)
    lg = jax.lax.dot_general(x1, w2_ref[...], _DN, **_HP) + b2_ref[...]
    m = jnp.max(lg, axis=1, keepdims=True)
    logp = lg - m - jnp.log(jnp.sum(jnp.exp(lg - m), axis=1, keepdims=True))

    def nrm(v):
        n = jnp.sqrt(jnp.sum(v * v, axis=1, keepdims=True))
        return v / jnp.maximum(n, 1e-12)

    scn = nrm(sp)
    fcn = nrm(fp)
    row = lax.broadcasted_iota(jnp.int32, (BB, BB), 0)
    col = lax.broadcasted_iota(jnp.int32, (BB, BB), 1)
    off = jnp.where(row == col, 0.0, 1.0)
    sf = jax.lax.dot_general(scn, fcn, _DT, **_HP) / TEMP
    fs = jax.lax.dot_general(fcn, scn, _DT, **_HP) / TEMP
    ssg = jax.lax.dot_general(scn, scn, _DT, **_HP) / TEMP * NEG_W * off
    ffg = jax.lax.dot_general(fcn, fcn, _DT, **_HP) / TEMP * NEG_W * off
    diag = jnp.sum(jnp.where(row == col, sf, 0.0), axis=1, keepdims=True)

    def lse2(a, bmat):
        mm = jnp.maximum(jnp.max(a, axis=1, keepdims=True),
                         jnp.max(bmat, axis=1, keepdims=True))
        ssum = (jnp.sum(jnp.exp(a - mm), axis=1, keepdims=True)
                + jnp.sum(jnp.exp(bmat - mm), axis=1, keepdims=True))
        return mm + jnp.log(ssum)

    loss_i = jnp.sum(lse2(sf, ssg) - diag) / BB
    loss_t = jnp.sum(lse2(fs, ffg) - diag) / BB
    out_ref[...] = logp + (loss_i + loss_t) * 0.5


def _tc_head(pooled, W1, b1, W2, b2):
    return pl.pallas_call(
        _head_body,
        out_shape=jax.ShapeDtypeStruct((BB, NCLS), jnp.float32),
    )(pooled[:BB], pooled[BB:], W1[:128], W1[128:], b1, W2, b2)


def _pack_edges(ei, z_off):
    src = jnp.concatenate([ei[0] + z_off,
                           jnp.full((EPAD - EE,), z_off, jnp.int32)])
    dst = jnp.concatenate([ei[1], jnp.full((EPAD - EE,), TRASH, jnp.int32)])
    return src.reshape(NTILES * NCH, CH), dst.reshape(NTILES * NCH, CH)


def kernel(sc_x, sc_edge_index, batch, fc_x, fc_edge_index, Wsc0, bsc0, Wsc1,
           bsc1, Wfc0, bfc0, Wfc1, bfc1, W1, b1, W2, b2):
    pad = jnp.zeros((NP_ - NN, DD), jnp.float32)
    xs = jnp.concatenate([sc_x, pad, fc_x, pad], axis=0)
    ssrc, sdst = _pack_edges(sc_edge_index, 0)
    fsrc, fdst = _pack_edges(fc_edge_index, NP_)
    src2d = jnp.concatenate([ssrc, fsrc], axis=0)
    dst2d = jnp.concatenate([sdst, fdst], axis=0)
    batch2d = jnp.concatenate(
        [batch, jnp.full((NP_ - NN,), BB, jnp.int32)]).reshape(NP_, 1)

    w0s = jnp.stack([Wsc0, Wfc0])
    w1s = jnp.stack([Wsc1, Wfc1])
    b0s = jnp.stack([bsc0.reshape(1, DD), bfc0.reshape(1, DD)])
    b1s = jnp.stack([bsc1.reshape(1, DD), bfc1.reshape(1, DD)])

    cnt = _sc_degree_kernel()(dst2d)
    dinv = _tc_dinv(cnt)
    za, zb = _tc_prep(xs, w0s, dinv)
    p0 = _sc_spmm_kernel()(za, zb, src2d, dst2d)
    za1, zb1 = _tc_mid(p0, za, zb, dinv, b0s, w1s)
    p1 = _sc_spmm_kernel()(za1, zb1, src2d, dst2d)
    ph0, ph1 = _tc_pool(p1, za1, zb1, dinv, b1s, batch2d)
    pooled = jnp.concatenate([ph0, ph1], axis=1)
    return _tc_head(pooled, W1, b1.reshape(1, DD), W2, b2.reshape(1, NCLS))
